# baseline (speedup 1.0000x reference)
import functools

import jax
import jax.numpy as jnp
from jax import lax
from jax.experimental import pallas as pl
from jax.experimental.pallas import tpu as pltpu
from jax.experimental.pallas import tpu_sc as plsc

NUM_CORES = 2
NUM_SUBCORES = 16
NUM_WORKERS = NUM_CORES * NUM_SUBCORES
SPAN = 3200
CHUNK = 80
NBUF = 8
DEPTH = 7


def kernel(input, node_segment):
    n = node_segment.shape[0]
    v, d = input.shape
    cps = SPAN // CHUNK
    full_workers = n // SPAN
    rem = n - full_workers * SPAN
    rem_chunks = rem // CHUNK
    assert SPAN % CHUNK == 0 and rem % CHUNK == 0
    assert SPAN % 8 == 0 and CHUNK % 8 == 0
    assert full_workers + (1 if rem else 0) == NUM_WORKERS
    assert min(cps, rem_chunks if rem else cps) >= NBUF

    idx = node_segment.astype(jnp.int32)
    mesh = plsc.VectorSubcoreMesh(core_axis_name="c", subcore_axis_name="s")

    @functools.partial(
        pl.kernel,
        out_type=jax.ShapeDtypeStruct((n, d), input.dtype),
        mesh=mesh,
        scratch_types=[
            pltpu.VMEM((SPAN,), jnp.int32),
            pltpu.VMEM((NBUF, CHUNK, d), jnp.float32),
            pltpu.VMEM_SHARED((v, d), jnp.float32),
            pltpu.SemaphoreType.DMA,
            pltpu.SemaphoreType.DMA((NBUF,)),
            pltpu.SemaphoreType.DMA((NBUF,)),
        ],
    )
    def gather_kernel(table_hbm, idx_hbm, out_hbm, idx_all, rows_v, table_sh,
                      sem_i, sem_g, sem_w):
        sid = lax.axis_index("s")
        wid = sid * NUM_CORES + lax.axis_index("c")
        base = wid * SPAN

        @pl.when(wid < full_workers)
        def _():
            pltpu.async_copy(idx_hbm.at[pl.ds(base, SPAN)],
                             idx_all.at[pl.ds(0, SPAN)], sem_i)

        if rem:
            @pl.when(wid == full_workers)
            def _():
                pltpu.async_copy(idx_hbm.at[pl.ds(base, rem)],
                                 idx_all.at[pl.ds(0, rem)], sem_i)

        rows_per_sub = v // NUM_SUBCORES
        assert rows_per_sub * NUM_SUBCORES == v
        pltpu.sync_copy(table_hbm.at[pl.ds(sid * rows_per_sub, rows_per_sub)],
                        table_sh.at[pl.ds(sid * rows_per_sub, rows_per_sub)])
        plsc.subcore_barrier()

        @pl.when(wid < full_workers)
        def _():
            pltpu.make_async_copy(idx_hbm.at[pl.ds(base, SPAN)],
                                  idx_all.at[pl.ds(0, SPAN)], sem_i).wait()

        if rem:
            @pl.when(wid == full_workers)
            def _():
                pltpu.make_async_copy(idx_hbm.at[pl.ds(base, rem)],
                                      idx_all.at[pl.ds(0, rem)], sem_i).wait()

        def guarded(k, fn):
            if k < (rem_chunks if rem else cps):
                fn()
            else:
                pl.when(wid < full_workers)(fn)

        def start_gather(k):
            b = k % NBUF
            pltpu.async_copy(
                table_sh.at[idx_all.at[pl.ds(k * CHUNK, CHUNK)]],
                rows_v.at[b], sem_g.at[b])

        def wait_write(k):
            b = k % NBUF
            pltpu.make_async_copy(
                rows_v.at[b], out_hbm.at[pl.ds(0, CHUNK)], sem_w.at[b]).wait()

        def finish_chunk(k):
            b = k % NBUF
            pltpu.make_async_copy(
                table_sh.at[idx_all.at[pl.ds(k * CHUNK, CHUNK)]],
                rows_v.at[b], sem_g.at[b]).wait()
            pltpu.async_copy(
                rows_v.at[b], out_hbm.at[pl.ds(base + k * CHUNK, CHUNK)],
                sem_w.at[b])

        for k in range(min(DEPTH, cps)):
            guarded(k, lambda k=k: start_gather(k))
        for k in range(cps):
            guarded(k, lambda k=k: finish_chunk(k))
            j = k + DEPTH
            if j < cps:
                def advance(j=j):
                    if j >= NBUF:
                        wait_write(j - NBUF)
                    start_gather(j)
                guarded(j, advance)

        for b in range(NBUF):
            wait_write(b)

    return gather_kernel(input, idx)


# speedup vs baseline: 1.0044x; 1.0044x over previous
import functools

import jax
import jax.numpy as jnp
from jax import lax
from jax.experimental import pallas as pl
from jax.experimental.pallas import tpu as pltpu
from jax.experimental.pallas import tpu_sc as plsc

NUM_CORES = 2
NUM_SUBCORES = 16
NUM_WORKERS = NUM_CORES * NUM_SUBCORES
SPAN = 3200
CHUNK = 80
NBUF = 6
DEPTH = 5


def kernel(input, node_segment):
    n = node_segment.shape[0]
    v, d = input.shape
    cps = SPAN // CHUNK
    full_workers = n // SPAN
    rem = n - full_workers * SPAN
    rem_chunks = rem // CHUNK
    assert SPAN % CHUNK == 0 and rem % CHUNK == 0
    assert SPAN % 8 == 0 and CHUNK % 8 == 0
    assert full_workers + (1 if rem else 0) == NUM_WORKERS
    assert min(cps, rem_chunks if rem else cps) >= NBUF

    idx = node_segment.astype(jnp.int32)
    mesh = plsc.VectorSubcoreMesh(core_axis_name="c", subcore_axis_name="s")

    @functools.partial(
        pl.kernel,
        out_type=jax.ShapeDtypeStruct((n, d), input.dtype),
        mesh=mesh,
        scratch_types=[
            pltpu.VMEM((SPAN,), jnp.int32),
            pltpu.VMEM((NBUF, CHUNK, d), jnp.float32),
            pltpu.VMEM_SHARED((v, d), jnp.float32),
            pltpu.SemaphoreType.DMA,
            pltpu.SemaphoreType.DMA((NBUF,)),
            pltpu.SemaphoreType.DMA((NBUF,)),
        ],
    )
    def gather_kernel(table_hbm, idx_hbm, out_hbm, idx_all, rows_v, table_sh,
                      sem_i, sem_g, sem_w):
        sid = lax.axis_index("s")
        wid = sid * NUM_CORES + lax.axis_index("c")
        base = wid * SPAN

        @pl.when(wid < full_workers)
        def _():
            pltpu.async_copy(idx_hbm.at[pl.ds(base, SPAN)],
                             idx_all.at[pl.ds(0, SPAN)], sem_i)

        if rem:
            @pl.when(wid == full_workers)
            def _():
                pltpu.async_copy(idx_hbm.at[pl.ds(base, rem)],
                                 idx_all.at[pl.ds(0, rem)], sem_i)

        rows_per_sub = v // NUM_SUBCORES
        assert rows_per_sub * NUM_SUBCORES == v
        pltpu.sync_copy(table_hbm.at[pl.ds(sid * rows_per_sub, rows_per_sub)],
                        table_sh.at[pl.ds(sid * rows_per_sub, rows_per_sub)])
        plsc.subcore_barrier()

        @pl.when(wid < full_workers)
        def _():
            pltpu.make_async_copy(idx_hbm.at[pl.ds(base, SPAN)],
                                  idx_all.at[pl.ds(0, SPAN)], sem_i).wait()

        if rem:
            @pl.when(wid == full_workers)
            def _():
                pltpu.make_async_copy(idx_hbm.at[pl.ds(base, rem)],
                                      idx_all.at[pl.ds(0, rem)], sem_i).wait()

        def guarded(k, fn):
            if k < (rem_chunks if rem else cps):
                fn()
            else:
                pl.when(wid < full_workers)(fn)

        def start_gather(k):
            b = k % NBUF
            pltpu.async_copy(
                table_sh.at[idx_all.at[pl.ds(k * CHUNK, CHUNK)]],
                rows_v.at[b], sem_g.at[b])

        def wait_write(k):
            b = k % NBUF
            pltpu.make_async_copy(
                rows_v.at[b], out_hbm.at[pl.ds(0, CHUNK)], sem_w.at[b]).wait()

        def finish_chunk(k):
            b = k % NBUF
            pltpu.make_async_copy(
                table_sh.at[idx_all.at[pl.ds(k * CHUNK, CHUNK)]],
                rows_v.at[b], sem_g.at[b]).wait()
            pltpu.async_copy(
                rows_v.at[b], out_hbm.at[pl.ds(base + k * CHUNK, CHUNK)],
                sem_w.at[b])

        for k in range(min(DEPTH, cps)):
            guarded(k, lambda k=k: start_gather(k))
        for k in range(cps):
            guarded(k, lambda k=k: finish_chunk(k))
            j = k + DEPTH
            if j < cps:
                def advance(j=j):
                    if j >= NBUF:
                        wait_write(j - NBUF)
                    start_gather(j)
                guarded(j, advance)

        for b in range(NBUF):
            wait_write(b)

    return gather_kernel(input, idx)
